# Initial kernel scaffold; baseline (speedup 1.0000x reference)
import jax
import jax.numpy as jnp
from jax.experimental import pallas as pl


def kernel(Ra, idx_i, idx_j, offsets):
    raise NotImplementedError("write your pallas kernel here")


# speedup vs baseline: 5.7328x; 5.7328x over previous
import jax
import jax.numpy as jnp
from jax import lax
from jax.experimental import pallas as pl
from jax.experimental.pallas import tpu as pltpu
from jax.experimental.pallas import tpu_sc as plsc

NC, NS = 2, 16
NW = NC * NS
C = 1600
G = 40
NG = C // G
LANES = 16
STEPS = C // LANES


def _body(xs_hbm, ys_hbm, zs_hbm, ii_hbm, jj_hbm, off_hbm, out_hbm,
          xs_sh, ys_sh, zs_sh, ii_v, jj_v,
          xi_v, yi_v, zi_v, xj_v, yj_v, zj_v,
          off_v, out_v, gsem):
  per_w_rows = ii_hbm.shape[0] // NW
  n_chunks = per_w_rows // NG

  cid = lax.axis_index("c")
  sid = lax.axis_index("s")
  wid = sid * NC + cid

  @pl.when(sid == 0)
  def _():
    pltpu.sync_copy(xs_hbm, xs_sh)
    pltpu.sync_copy(ys_hbm, ys_sh)
    pltpu.sync_copy(zs_hbm, zs_sh)
  plsc.subcore_barrier()

  tri = 3 * lax.iota(jnp.int32, LANES)

  def chunk(g, carry):
    row_base = wid * per_w_rows + g * NG
    base = row_base * G
    pltpu.sync_copy(ii_hbm.at[pl.ds(row_base, NG)], ii_v)
    pltpu.sync_copy(jj_hbm.at[pl.ds(row_base, NG)], jj_v)
    pltpu.sync_copy(off_hbm.at[pl.ds(3 * base, 3 * C)], off_v)

    def fire(k, c2):
      dst = pl.ds(k * G, G)
      pltpu.async_copy(xs_sh.at[ii_v.at[k]], xi_v.at[dst], gsem)
      pltpu.async_copy(ys_sh.at[ii_v.at[k]], yi_v.at[dst], gsem)
      pltpu.async_copy(zs_sh.at[ii_v.at[k]], zi_v.at[dst], gsem)
      pltpu.async_copy(xs_sh.at[jj_v.at[k]], xj_v.at[dst], gsem)
      pltpu.async_copy(ys_sh.at[jj_v.at[k]], yj_v.at[dst], gsem)
      pltpu.async_copy(zs_sh.at[jj_v.at[k]], zj_v.at[dst], gsem)
      return c2
    lax.fori_loop(0, NG, fire, 0)
    for buf in (xi_v, yi_v, zi_v, xj_v, yj_v, zj_v):
      pltpu.make_async_copy(xs_hbm.at[pl.ds(0, C)], buf, gsem).wait()

    def step(s, c2):
      sl = pl.ds(s * LANES, LANES)
      base3 = s * (3 * LANES) + tri
      ox = plsc.load_gather(off_v, [base3])
      oy = plsc.load_gather(off_v, [base3 + 1])
      oz = plsc.load_gather(off_v, [base3 + 2])
      dx = xi_v[sl] - xj_v[sl] - ox
      dy = yi_v[sl] - yj_v[sl] - oy
      dz = zi_v[sl] - zj_v[sl] - oz
      d2 = jnp.maximum(dx * dx + dy * dy + dz * dz, 0.0)
      seed = plsc.bitcast(0x5F3759DF - (plsc.bitcast(d2, jnp.int32) >> 1),
                          jnp.float32)
      h = 0.5 * d2
      r = seed * (1.5 - h * seed * seed)
      r = r * (1.5 - h * r * r)
      out_v[sl] = d2 * r
      return c2
    lax.fori_loop(0, STEPS, step, 0)

    pltpu.sync_copy(out_v, out_hbm.at[pl.ds(base, C)])
    return carry

  lax.fori_loop(0, n_chunks, chunk, 0)


def kernel(Ra, idx_i, idx_j, offsets):
  n = Ra.shape[0]
  e = idx_i.shape[0]
  xs = Ra[:, 0]
  ys = Ra[:, 1]
  zs = Ra[:, 2]
  ii2 = idx_i.astype(jnp.int32).reshape(e // G, G)
  jj2 = idx_j.astype(jnp.int32).reshape(e // G, G)

  mesh = plsc.VectorSubcoreMesh(core_axis_name="c", subcore_axis_name="s")
  run = pl.kernel(
      _body,
      out_type=jax.ShapeDtypeStruct((e,), jnp.float32),
      mesh=mesh,
      compiler_params=pltpu.CompilerParams(needs_layout_passes=False),
      scratch_types=[
          pltpu.VMEM_SHARED((n,), jnp.float32),
          pltpu.VMEM_SHARED((n,), jnp.float32),
          pltpu.VMEM_SHARED((n,), jnp.float32),
          pltpu.VMEM((NG, G), jnp.int32),
          pltpu.VMEM((NG, G), jnp.int32),
          pltpu.VMEM((C,), jnp.float32),
          pltpu.VMEM((C,), jnp.float32),
          pltpu.VMEM((C,), jnp.float32),
          pltpu.VMEM((C,), jnp.float32),
          pltpu.VMEM((C,), jnp.float32),
          pltpu.VMEM((C,), jnp.float32),
          pltpu.VMEM((3 * C,), jnp.float32),
          pltpu.VMEM((C,), jnp.float32),
          pltpu.SemaphoreType.DMA,
      ],
  )
  return run(xs, ys, zs, ii2, jj2, offsets.reshape(3 * e))


# speedup vs baseline: 5.8827x; 1.0262x over previous
import jax
import jax.numpy as jnp
from jax import lax
from jax.experimental import pallas as pl
from jax.experimental.pallas import tpu as pltpu
from jax.experimental.pallas import tpu_sc as plsc

NC, NS = 2, 16
NW = NC * NS
C = 1600
G = 40
NG = C // G
LANES = 16
STEPS = C // LANES
OUT_BYTES = C * 4


def _body(xs_hbm, ys_hbm, zs_hbm, ii_hbm, jj_hbm, off_hbm, out_hbm,
          xs_sh, ys_sh, zs_sh, *bufs):
  per_w_rows = ii_hbm.shape[0] // NW
  n_chunks = per_w_rows // NG

  cid = lax.axis_index("c")
  sid = lax.axis_index("s")
  wid = sid * NC + cid

  (ii_a, jj_a, off_a, xi_a, yi_a, zi_a, xj_a, yj_a, zj_a, out_a,
   lsem_a, gsem_a, osem_a,
   ii_b, jj_b, off_b, xi_b, yi_b, zi_b, xj_b, yj_b, zj_b, out_b,
   lsem_b, gsem_b, osem_b) = bufs
  A = (ii_a, jj_a, off_a, (xi_a, yi_a, zi_a), (xj_a, yj_a, zj_a), out_a,
       lsem_a, gsem_a, osem_a)
  B = (ii_b, jj_b, off_b, (xi_b, yi_b, zi_b), (xj_b, yj_b, zj_b), out_b,
       lsem_b, gsem_b, osem_b)

  @pl.when(sid == 0)
  def _():
    pltpu.sync_copy(xs_hbm, xs_sh)
    pltpu.sync_copy(ys_hbm, ys_sh)
    pltpu.sync_copy(zs_hbm, zs_sh)
  plsc.subcore_barrier()

  tri = 3 * lax.iota(jnp.int32, LANES)
  tabs = (xs_sh, ys_sh, zs_sh)

  def row_base(g):
    return wid * per_w_rows + g * NG

  def lin_start(g, S):
    ii_v, jj_v, off_v, _, _, _, lsem, _, _ = S
    @pl.when(g < n_chunks)
    def _():
      rb = row_base(g)
      pltpu.async_copy(ii_hbm.at[pl.ds(rb, NG)], ii_v, lsem)
      pltpu.async_copy(jj_hbm.at[pl.ds(rb, NG)], jj_v, lsem)
      pltpu.async_copy(off_hbm.at[pl.ds(3 * rb * G, 3 * C)], off_v, lsem)

  def lin_wait(S):
    ii_v, jj_v, off_v, _, _, _, lsem, _, _ = S
    pltpu.make_async_copy(ii_hbm.at[pl.ds(0, NG)], ii_v, lsem).wait()
    pltpu.make_async_copy(jj_hbm.at[pl.ds(0, NG)], jj_v, lsem).wait()
    pltpu.make_async_copy(off_hbm.at[pl.ds(0, 3 * C)], off_v, lsem).wait()

  def gather_fire(S):
    ii_v, jj_v, _, ri, rj, _, _, gsem, _ = S
    def fire(k, c2):
      dst = pl.ds(k * G, G)
      for t in range(3):
        pltpu.async_copy(tabs[t].at[ii_v.at[k]], ri[t].at[dst], gsem)
        pltpu.async_copy(tabs[t].at[jj_v.at[k]], rj[t].at[dst], gsem)
      return c2
    lax.fori_loop(0, NG, fire, 0)

  def gather_drain(S):
    _, _, _, ri, rj, _, _, gsem, _ = S
    for buf in (*ri, *rj):
      pltpu.make_async_copy(xs_hbm.at[pl.ds(0, C)], buf, gsem).wait()

  def out_wait(S):
    out_v, osem = S[5], S[8]
    pltpu.make_async_copy(out_hbm.at[pl.ds(0, C)], out_v, osem).wait()

  def compute_store(g, S):
    _, _, off_v, (xi_v, yi_v, zi_v), (xj_v, yj_v, zj_v), out_v, _, _, osem = S
    def step(s, c2):
      sl = pl.ds(s * LANES, LANES)
      base3 = s * (3 * LANES) + tri
      ox = plsc.load_gather(off_v, [base3])
      oy = plsc.load_gather(off_v, [base3 + 1])
      oz = plsc.load_gather(off_v, [base3 + 2])
      dx = xi_v[sl] - xj_v[sl] - ox
      dy = yi_v[sl] - yj_v[sl] - oy
      dz = zi_v[sl] - zj_v[sl] - oz
      d2 = jnp.maximum(dx * dx + dy * dy + dz * dz, 0.0)
      seed = plsc.bitcast(0x5F3759DF - (plsc.bitcast(d2, jnp.int32) >> 1),
                          jnp.float32)
      h = 0.5 * d2
      r = seed * (1.5 - h * seed * seed)
      r = r * (1.5 - h * r * r)
      out_v[sl] = d2 * r
      return c2
    lax.fori_loop(0, STEPS, step, 0)
    pltpu.async_copy(out_v, out_hbm.at[pl.ds(row_base(g) * G, C)], osem)

  pltpu.async_copy(out_hbm.at[pl.ds(0, C)], A[5], A[8])
  pltpu.async_copy(out_hbm.at[pl.ds(0, C)], B[5], B[8])

  lin_start(0, A)
  lin_wait(A)
  gather_fire(A)
  lin_start(1, B)

  def pair(t, carry):
    g = 2 * t
    gather_drain(A)
    lin_wait(B)
    gather_fire(B)
    out_wait(A)
    compute_store(g, A)
    lin_start(g + 2, A)
    gather_drain(B)
    @pl.when(g + 2 < n_chunks)
    def _():
      lin_wait(A)
      gather_fire(A)
    out_wait(B)
    compute_store(g + 1, B)
    lin_start(g + 3, B)
    return carry

  lax.fori_loop(0, (n_chunks - 1) // 2, pair, 0)

  gather_drain(A)
  out_wait(A)
  compute_store(n_chunks - 1, A)
  out_wait(A)
  out_wait(B)


def kernel(Ra, idx_i, idx_j, offsets):
  n = Ra.shape[0]
  e = idx_i.shape[0]
  xs = Ra[:, 0]
  ys = Ra[:, 1]
  zs = Ra[:, 2]
  ii2 = idx_i.astype(jnp.int32).reshape(e // G, G)
  jj2 = idx_j.astype(jnp.int32).reshape(e // G, G)

  mesh = plsc.VectorSubcoreMesh(core_axis_name="c", subcore_axis_name="s")
  buf_set = [
      pltpu.VMEM((NG, G), jnp.int32),
      pltpu.VMEM((NG, G), jnp.int32),
      pltpu.VMEM((3 * C,), jnp.float32),
      pltpu.VMEM((C,), jnp.float32),
      pltpu.VMEM((C,), jnp.float32),
      pltpu.VMEM((C,), jnp.float32),
      pltpu.VMEM((C,), jnp.float32),
      pltpu.VMEM((C,), jnp.float32),
      pltpu.VMEM((C,), jnp.float32),
      pltpu.VMEM((C,), jnp.float32),
      pltpu.SemaphoreType.DMA,
      pltpu.SemaphoreType.DMA,
      pltpu.SemaphoreType.DMA,
  ]
  run = pl.kernel(
      _body,
      out_type=jax.ShapeDtypeStruct((e,), jnp.float32),
      mesh=mesh,
      compiler_params=pltpu.CompilerParams(needs_layout_passes=False),
      scratch_types=[
          pltpu.VMEM_SHARED((n,), jnp.float32),
          pltpu.VMEM_SHARED((n,), jnp.float32),
          pltpu.VMEM_SHARED((n,), jnp.float32),
          *buf_set,
          *buf_set,
      ],
  )
  return run(xs, ys, zs, ii2, jj2, offsets.reshape(3 * e))


# speedup vs baseline: 5.9167x; 1.0058x over previous
import jax
import jax.numpy as jnp
from jax import lax
from jax.experimental import pallas as pl
from jax.experimental.pallas import tpu as pltpu
from jax.experimental.pallas import tpu_sc as plsc

NC, NS = 2, 16
NW = NC * NS
C = 1600
G = 40
NG = C // G
LANES = 16
STEPS = C // LANES


def _body(xs_hbm, ys_hbm, zs_hbm, ii_hbm, jj_hbm, off_hbm, out_hbm,
          xs_sh, ys_sh, zs_sh, *bufs):
  per_w = ii_hbm.shape[0] // NW
  n_chunks = per_w // C

  cid = lax.axis_index("c")
  sid = lax.axis_index("s")
  wid = sid * NC + cid

  (ii_a, jj_a, off_a, xi_a, yi_a, zi_a, xj_a, yj_a, zj_a, out_a,
   lsem_a, gsem_a, osem_a,
   ii_b, jj_b, off_b, xi_b, yi_b, zi_b, xj_b, yj_b, zj_b, out_b,
   lsem_b, gsem_b, osem_b) = bufs
  A = (ii_a, jj_a, off_a, (xi_a, yi_a, zi_a), (xj_a, yj_a, zj_a), out_a,
       lsem_a, gsem_a, osem_a)
  B = (ii_b, jj_b, off_b, (xi_b, yi_b, zi_b), (xj_b, yj_b, zj_b), out_b,
       lsem_b, gsem_b, osem_b)

  @pl.when(sid == 0)
  def _():
    pltpu.sync_copy(xs_hbm, xs_sh)
    pltpu.sync_copy(ys_hbm, ys_sh)
    pltpu.sync_copy(zs_hbm, zs_sh)
  plsc.subcore_barrier()

  tri = 3 * lax.iota(jnp.int32, LANES)
  tabs = (xs_sh, ys_sh, zs_sh)

  def chunk_base(g):
    return wid * per_w + g * C

  def lin_start(g, S):
    ii_v, jj_v, off_v, _, _, _, lsem, _, _ = S
    @pl.when(g < n_chunks)
    def _():
      base = chunk_base(g)
      pltpu.async_copy(ii_hbm.at[pl.ds(base, C)], ii_v, lsem)
      pltpu.async_copy(jj_hbm.at[pl.ds(base, C)], jj_v, lsem)
      pltpu.async_copy(off_hbm.at[pl.ds(3 * base, 3 * C)], off_v, lsem)

  def lin_wait(S):
    ii_v, jj_v, off_v, _, _, _, lsem, _, _ = S
    pltpu.make_async_copy(ii_hbm.at[pl.ds(0, C)], ii_v, lsem).wait()
    pltpu.make_async_copy(jj_hbm.at[pl.ds(0, C)], jj_v, lsem).wait()
    pltpu.make_async_copy(off_hbm.at[pl.ds(0, 3 * C)], off_v, lsem).wait()

  def gather_fire(S):
    ii_v, jj_v, _, ri, rj, _, _, gsem, _ = S
    def fire(k, c2):
      sl = pl.ds(k * G, G)
      for t in range(3):
        pltpu.async_copy(tabs[t].at[ii_v.at[sl]], ri[t].at[sl], gsem)
        pltpu.async_copy(tabs[t].at[jj_v.at[sl]], rj[t].at[sl], gsem)
      return c2
    lax.fori_loop(0, NG, fire, 0)

  def gather_drain(S):
    _, _, _, ri, rj, _, _, gsem, _ = S
    for buf in (*ri, *rj):
      pltpu.make_async_copy(xs_hbm.at[pl.ds(0, C)], buf, gsem).wait()

  def out_wait(S):
    out_v, osem = S[5], S[8]
    pltpu.make_async_copy(out_hbm.at[pl.ds(0, C)], out_v, osem).wait()

  def compute_store(g, S):
    _, _, off_v, (xi_v, yi_v, zi_v), (xj_v, yj_v, zj_v), out_v, _, _, osem = S
    def step(s, c2):
      sl = pl.ds(s * LANES, LANES)
      base3 = s * (3 * LANES) + tri
      ox = plsc.load_gather(off_v, [base3])
      oy = plsc.load_gather(off_v, [base3 + 1])
      oz = plsc.load_gather(off_v, [base3 + 2])
      dx = xi_v[sl] - xj_v[sl] - ox
      dy = yi_v[sl] - yj_v[sl] - oy
      dz = zi_v[sl] - zj_v[sl] - oz
      d2 = jnp.maximum(dx * dx + dy * dy + dz * dz, 0.0)
      seed = plsc.bitcast(0x5F3759DF - (plsc.bitcast(d2, jnp.int32) >> 1),
                          jnp.float32)
      h = 0.5 * d2
      r = seed * (1.5 - h * seed * seed)
      r = r * (1.5 - h * r * r)
      out_v[sl] = d2 * r
      return c2
    lax.fori_loop(0, STEPS, step, 0)
    pltpu.async_copy(out_v, out_hbm.at[pl.ds(chunk_base(g), C)], osem)

  pltpu.async_copy(out_hbm.at[pl.ds(0, C)], A[5], A[8])
  pltpu.async_copy(out_hbm.at[pl.ds(0, C)], B[5], B[8])

  lin_start(0, A)
  lin_wait(A)
  gather_fire(A)
  lin_start(1, B)

  def pair(t, carry):
    g = 2 * t
    gather_drain(A)
    lin_wait(B)
    gather_fire(B)
    out_wait(A)
    compute_store(g, A)
    lin_start(g + 2, A)
    gather_drain(B)
    @pl.when(g + 2 < n_chunks)
    def _():
      lin_wait(A)
      gather_fire(A)
    out_wait(B)
    compute_store(g + 1, B)
    lin_start(g + 3, B)
    return carry

  lax.fori_loop(0, (n_chunks - 1) // 2, pair, 0)

  gather_drain(A)
  out_wait(A)
  compute_store(n_chunks - 1, A)
  out_wait(A)
  out_wait(B)


def kernel(Ra, idx_i, idx_j, offsets):
  n = Ra.shape[0]
  e = idx_i.shape[0]
  xs = Ra[:, 0]
  ys = Ra[:, 1]
  zs = Ra[:, 2]

  mesh = plsc.VectorSubcoreMesh(core_axis_name="c", subcore_axis_name="s")
  buf_set = [
      pltpu.VMEM((C,), jnp.int32),
      pltpu.VMEM((C,), jnp.int32),
      pltpu.VMEM((3 * C,), jnp.float32),
      pltpu.VMEM((C,), jnp.float32),
      pltpu.VMEM((C,), jnp.float32),
      pltpu.VMEM((C,), jnp.float32),
      pltpu.VMEM((C,), jnp.float32),
      pltpu.VMEM((C,), jnp.float32),
      pltpu.VMEM((C,), jnp.float32),
      pltpu.VMEM((C,), jnp.float32),
      pltpu.SemaphoreType.DMA,
      pltpu.SemaphoreType.DMA,
      pltpu.SemaphoreType.DMA,
  ]
  run = pl.kernel(
      _body,
      out_type=jax.ShapeDtypeStruct((e,), jnp.float32),
      mesh=mesh,
      compiler_params=pltpu.CompilerParams(needs_layout_passes=False),
      scratch_types=[
          pltpu.VMEM_SHARED((n,), jnp.float32),
          pltpu.VMEM_SHARED((n,), jnp.float32),
          pltpu.VMEM_SHARED((n,), jnp.float32),
          *buf_set,
          *buf_set,
      ],
  )
  return run(xs, ys, zs, idx_i.astype(jnp.int32), idx_j.astype(jnp.int32),
             offsets.reshape(3 * e))
